# Initial kernel scaffold; baseline (speedup 1.0000x reference)
import jax
import jax.numpy as jnp
from jax.experimental import pallas as pl


def kernel(x, edge_index, W1, W2, W3, b3, g1, be1, g2, be2):
    raise NotImplementedError("write your pallas kernel here")


# speedup vs baseline: 12.7053x; 12.7053x over previous
import functools

import jax
import jax.numpy as jnp
from jax import lax
from jax.experimental import pallas as pl
from jax.experimental.pallas import tpu as pltpu
from jax.experimental.pallas import tpu_sc as plsc

_N = 10000
_E = 320000
_DH = 128
_DOUT = 40
_DOUT_PAD = 128
_EPS = 1e-5

_NC = 2
_NS = 16
_NW = _NC * _NS
_C = 80
_EPT = _E // _NW
_NCH = _EPT // _C
_NP = 10240
_RPT = _NP // _NS
_ZR = 128

_ROW_BLK = 1000


def _zero_vmem_2d(ref, rows, d):
    z = jnp.zeros((16,), jnp.float32)

    def body(i, _):
        for j in range(d // 16):
            ref[i, pl.ds(j * 16, 16)] = z
        return 0

    lax.fori_loop(0, rows, body, 0)


def _make_prop(d):
    mesh = plsc.VectorSubcoreMesh(core_axis_name="c", subcore_axis_name="s")

    @functools.partial(
        pl.kernel,
        out_type=jax.ShapeDtypeStruct((_NC, _NP, d), jnp.float32),
        mesh=mesh,
        scratch_types=[
            pltpu.VMEM((_NCH, _C), jnp.int32),
            pltpu.VMEM((_NCH, _C), jnp.int32),
            pltpu.VMEM((_C, d), jnp.float32),
            pltpu.VMEM_SHARED((_NP, d), jnp.float32),
            pltpu.SemaphoreType.DMA,
        ],
    )
    def prop(h_hbm, src_hbm, dst_hbm, out_hbm, src_v, dst_v, rows_v,
             acc_sh, sem):
        cid = lax.axis_index("c")
        sid = lax.axis_index("s")
        tid = cid * _NS + sid

        pltpu.sync_copy(src_hbm.at[tid], src_v)
        pltpu.sync_copy(dst_hbm.at[tid], dst_v)

        _zero_vmem_2d(rows_v, _C, d)
        for r in range(_RPT // _C):
            pltpu.sync_copy(rows_v, acc_sh.at[pl.ds(sid * _RPT + r * _C, _C)])
        plsc.subcore_barrier()

        def body(k, _):
            pltpu.async_copy(h_hbm.at[src_v.at[k]], rows_v, sem).wait()
            pltpu.sync_copy(rows_v, acc_sh.at[dst_v.at[k]], add=True)
            return 0

        lax.fori_loop(0, _NCH, body, 0)

        plsc.subcore_barrier()
        pltpu.sync_copy(acc_sh.at[pl.ds(sid * _RPT, _RPT)],
                        out_hbm.at[cid, pl.ds(sid * _RPT, _RPT)])

    return prop


def _make_deg():
    d = 128
    mesh = plsc.VectorSubcoreMesh(core_axis_name="c", subcore_axis_name="s")

    @functools.partial(
        pl.kernel,
        out_type=jax.ShapeDtypeStruct((_NC, _NP, d), jnp.float32),
        mesh=mesh,
        scratch_types=[
            pltpu.VMEM((_NCH, _C), jnp.int32),
            pltpu.VMEM((_C, d), jnp.float32),
            pltpu.VMEM_SHARED((_NP, d), jnp.float32),
        ],
    )
    def deg(dst_hbm, out_hbm, dst_v, ones_v, acc_sh):
        cid = lax.axis_index("c")
        sid = lax.axis_index("s")
        tid = cid * _NS + sid

        pltpu.sync_copy(dst_hbm.at[tid], dst_v)

        _zero_vmem_2d(ones_v, _C, d)
        for r in range(_RPT // _C):
            pltpu.sync_copy(ones_v, acc_sh.at[pl.ds(sid * _RPT + r * _C, _C)])

        one = jnp.ones((16,), jnp.float32)

        def fill(i, _):
            for j in range(d // 16):
                ones_v[i, pl.ds(j * 16, 16)] = one
            return 0

        lax.fori_loop(0, _C, fill, 0)
        plsc.subcore_barrier()

        def body(k, _):
            pltpu.sync_copy(ones_v, acc_sh.at[dst_v.at[k]], add=True)
            return 0

        lax.fori_loop(0, _NCH, body, 0)

        plsc.subcore_barrier()
        pltpu.sync_copy(acc_sh.at[pl.ds(sid * _RPT, _RPT)],
                        out_hbm.at[cid, pl.ds(sid * _RPT, _RPT)])

    return deg


_prop128 = _make_prop(_DH)
_deg_sc = _make_deg()


_BN_SCALE = 1.0 / (1.0 + _EPS) ** 0.5


def _mm_body(x_ref, w_ref, o_ref):
    o_ref[...] = jnp.dot(x_ref[...], w_ref[...],
                         preferred_element_type=jnp.float32)


def _mm1(x, w):
    return pl.pallas_call(
        _mm_body,
        grid=(_N // _ROW_BLK,),
        in_specs=[
            pl.BlockSpec((_ROW_BLK, _DH), lambda i: (i, 0)),
            pl.BlockSpec((_DH, _DH), lambda i: (0, 0)),
        ],
        out_specs=pl.BlockSpec((_ROW_BLK, _DH), lambda i: (i, 0)),
        out_shape=jax.ShapeDtypeStruct((_N, _DH), jnp.float32),
    )(x, w)


def _scale1_body(deg_ref, h_ref, dinv_ref, hp_ref, self_ref):
    deg = deg_ref[0, :, 0] + deg_ref[1, :, 0] + 2.0
    dinv = lax.rsqrt(deg)[:, None]
    h = h_ref[...]
    dinv_ref[...] = jnp.broadcast_to(dinv, h.shape)
    hp_ref[...] = dinv * h
    self_ref[...] = (2.0 * dinv * dinv) * h


def _scale1(deg16, h1):
    return pl.pallas_call(
        _scale1_body,
        grid=(_N // _ROW_BLK,),
        in_specs=[
            pl.BlockSpec((_NC, _ROW_BLK, _DH), lambda i: (0, i, 0)),
            pl.BlockSpec((_ROW_BLK, _DH), lambda i: (i, 0)),
        ],
        out_specs=[
            pl.BlockSpec((_ROW_BLK, _DH), lambda i: (i, 0)),
            pl.BlockSpec((_ROW_BLK, _DH), lambda i: (i, 0)),
            pl.BlockSpec((_ROW_BLK, _DH), lambda i: (i, 0)),
        ],
        out_shape=[
            jax.ShapeDtypeStruct((_N, _DH), jnp.float32),
            jax.ShapeDtypeStruct((_N, _DH), jnp.float32),
            jax.ShapeDtypeStruct((_N, _DH), jnp.float32),
        ],
    )(deg16, h1)


def _fuse_body(a_ref, self_ref, dinv_ref, w_ref, g_ref, be_ref,
               hp_ref, selfn_ref, *, dout):
    dinv = dinv_ref[...]
    t = dinv * (a_ref[0] + a_ref[1]) + self_ref[...]
    t = g_ref[...] * (t * _BN_SCALE) + be_ref[...]
    t = jnp.maximum(t, 0.0)
    h = jnp.dot(t, w_ref[...], preferred_element_type=jnp.float32)
    dv = dinv[:, :dout]
    hp_ref[...] = dv * h
    selfn_ref[...] = (2.0 * dv * dv) * h


def _fuse(a, self_prev, dinv, w, g, be, dout):
    return pl.pallas_call(
        functools.partial(_fuse_body, dout=dout),
        grid=(_N // _ROW_BLK,),
        in_specs=[
            pl.BlockSpec((_NC, _ROW_BLK, _DH), lambda i: (0, i, 0)),
            pl.BlockSpec((_ROW_BLK, _DH), lambda i: (i, 0)),
            pl.BlockSpec((_ROW_BLK, _DH), lambda i: (i, 0)),
            pl.BlockSpec((_DH, dout), lambda i: (0, 0)),
            pl.BlockSpec((1, _DH), lambda i: (0, 0)),
            pl.BlockSpec((1, _DH), lambda i: (0, 0)),
        ],
        out_specs=[
            pl.BlockSpec((_ROW_BLK, dout), lambda i: (i, 0)),
            pl.BlockSpec((_ROW_BLK, dout), lambda i: (i, 0)),
        ],
        out_shape=[
            jax.ShapeDtypeStruct((_N, dout), jnp.float32),
            jax.ShapeDtypeStruct((_N, dout), jnp.float32),
        ],
    )(a, self_prev, dinv, w, g, be)


def _final_body(a_ref, self_ref, dinv_ref, b_ref, o_ref):
    dv = dinv_ref[:, :_DOUT_PAD]
    t = dv * (a_ref[0] + a_ref[1]) + self_ref[...] + b_ref[...]
    col = lax.broadcasted_iota(jnp.int32, t.shape, 1)
    valid = col < _DOUT
    m = jnp.max(jnp.where(valid, t, -1e30), axis=1, keepdims=True)
    e = jnp.where(valid, jnp.exp(t - m), 0.0)
    lse = jnp.log(jnp.sum(e, axis=1, keepdims=True))
    o_ref[...] = t - m - lse


def _final(a3, self3, dinv, b3p):
    return pl.pallas_call(
        _final_body,
        grid=(_N // _ROW_BLK,),
        in_specs=[
            pl.BlockSpec((_NC, _ROW_BLK, _DOUT_PAD), lambda i: (0, i, 0)),
            pl.BlockSpec((_ROW_BLK, _DOUT_PAD), lambda i: (i, 0)),
            pl.BlockSpec((_ROW_BLK, _DH), lambda i: (i, 0)),
            pl.BlockSpec((1, _DOUT_PAD), lambda i: (0, 0)),
        ],
        out_specs=pl.BlockSpec((_ROW_BLK, _DOUT_PAD), lambda i: (i, 0)),
        out_shape=jax.ShapeDtypeStruct((_N, _DOUT_PAD), jnp.float32),
    )(a3, self3, dinv, b3p)


def kernel(x, edge_index, W1, W2, W3, b3, g1, be1, g2, be2):
    src = edge_index[0].reshape(_NW, _NCH, _C)
    dst = edge_index[1].reshape(_NW, _NCH, _C)
    w3p = jnp.pad(W3, ((0, 0), (0, _DOUT_PAD - _DOUT)))
    b3p = jnp.pad(b3, (0, _DOUT_PAD - _DOUT)).reshape(1, _DOUT_PAD)
    g1r = g1.reshape(1, _DH)
    be1r = be1.reshape(1, _DH)
    g2r = g2.reshape(1, _DH)
    be2r = be2.reshape(1, _DH)

    deg16 = _deg_sc(dst)
    h1 = _mm1(x, W1)
    dinv, h1p, self1 = _scale1(deg16, h1)
    a1 = _prop128(h1p, src, dst)
    h2p, self2 = _fuse(a1, self1, dinv, W2, g1r, be1r, _DH)
    a2 = _prop128(h2p, src, dst)
    h3p, self3 = _fuse(a2, self2, dinv, w3p, g2r, be2r, _DOUT_PAD)
    a3 = _prop128(h3p, src, dst)
    out = _final(a3, self3, dinv, b3p)
    return out[:, :_DOUT]
